# baseline (speedup 1.0000x reference)
import jax
import jax.numpy as jnp
from jax import lax
from jax.experimental import pallas as pl
from jax.experimental.pallas import tpu as pltpu
from jax.experimental.pallas import tpu_sc as plsc

N_SP = 64
N_AT = 100000
NCORE = 2
NSUB = 16
NW = NCORE * NSUB
LANES = 16
CHUNK = 3136
TAIL = N_AT - (NW - 1) * CHUNK
STEPS = CHUNK // LANES
ACCS = 4
TBL = N_SP + LANES


def _sc_body(idx_hbm, shifts_hbm, part_hbm, idx_v, tbl_v, vec_v, sem0, sem1):
    cid = lax.axis_index("c")
    sid = lax.axis_index("s")
    wid = cid * NSUB + sid
    base = wid * CHUNK
    is_tail = wid == NW - 1

    tbl_cp = pltpu.async_copy(shifts_hbm, tbl_v.at[pl.ds(0, N_SP)], sem0)
    tbl_v[pl.ds(N_SP, LANES)] = jnp.zeros((LANES,), jnp.float32)
    pad = jnp.full((LANES,), N_SP, jnp.int32)

    @pl.when(jnp.logical_not(is_tail))
    def _():
        pltpu.async_copy(idx_hbm.at[pl.ds(base, CHUNK)],
                         idx_v.at[pl.ds(0, CHUNK)], sem1).wait()

    @pl.when(is_tail)
    def _():
        pltpu.async_copy(idx_hbm.at[pl.ds(base, TAIL)],
                         idx_v.at[pl.ds(0, TAIL)], sem1).wait()
        for i in range(TAIL // LANES, STEPS):
            idx_v[pl.ds(i * LANES, LANES)] = pad

    tbl_cp.wait()
    zero = jnp.zeros((LANES,), jnp.float32)

    def step4(j, accs):
        a0, a1, a2, a3 = accs
        b = j * (ACCS * LANES)
        a0 = a0 + plsc.load_gather(tbl_v, [idx_v[pl.ds(b, LANES)]])
        a1 = a1 + plsc.load_gather(tbl_v, [idx_v[pl.ds(b + LANES, LANES)]])
        a2 = a2 + plsc.load_gather(tbl_v, [idx_v[pl.ds(b + 2 * LANES, LANES)]])
        a3 = a3 + plsc.load_gather(tbl_v, [idx_v[pl.ds(b + 3 * LANES, LANES)]])
        return a0, a1, a2, a3

    accs = lax.fori_loop(0, STEPS // ACCS, step4, (zero, zero, zero, zero),
                         unroll=2)
    vec_v[...] = (accs[0] + accs[1]) + (accs[2] + accs[3])
    pltpu.sync_copy(vec_v, part_hbm.at[wid])


def _tc_finish(part_ref, te_ref, ts_ref, out_ref):
    s = jnp.sum(part_ref[...]) + ts_ref[0, 0]
    out_ref[...] = te_ref[...] + s


@jax.jit
def _shift_sum(idx, shifts, te_row, ts):
    mesh = plsc.VectorSubcoreMesh(core_axis_name="c", subcore_axis_name="s",
                                  num_cores=NCORE)
    partials = pl.kernel(
        _sc_body,
        out_type=jax.ShapeDtypeStruct((NW, LANES), jnp.float32),
        mesh=mesh,
        compiler_params=pltpu.CompilerParams(needs_layout_passes=False),
        scratch_types=[
            pltpu.VMEM((CHUNK,), jnp.int32),
            pltpu.VMEM((TBL,), jnp.float32),
            pltpu.VMEM((LANES,), jnp.float32),
            pltpu.SemaphoreType.DMA,
            pltpu.SemaphoreType.DMA,
        ],
    )(idx, shifts)
    return pl.pallas_call(
        _tc_finish,
        out_shape=jax.ShapeDtypeStruct((1, LANES), jnp.float32),
    )(partials, te_row, ts)


def kernel(total_energy, species_index, shifts, total_shift):
    idx = species_index.astype(jnp.int32)
    te_row = total_energy.reshape(1, LANES)
    ts = total_shift.astype(jnp.float32).reshape(1, 1)
    out = _shift_sum(idx, shifts, te_row, ts)
    return out.reshape(total_energy.shape)


# speedup vs baseline: 1.0587x; 1.0587x over previous
import jax
import jax.numpy as jnp
from jax import lax
from jax.experimental import pallas as pl
from jax.experimental.pallas import tpu as pltpu
from jax.experimental.pallas import tpu_sc as plsc

N_SP = 64
N_AT = 100000
NSUB = 16
LANES = 16
CHUNK = 6256
TAIL = N_AT - (NSUB - 1) * CHUNK
ALLOC = 6272
STEPS = ALLOC // LANES
ACCS = 4
TBL = N_SP + LANES


def _sc_body(idx_hbm, shifts_hbm, part_hbm, idx_v, tbl_v, vec_v, sem0, sem1):
    sid = lax.axis_index("s")
    base = sid * CHUNK
    is_tail = sid == NSUB - 1

    tbl_cp = pltpu.async_copy(shifts_hbm, tbl_v.at[pl.ds(0, N_SP)], sem0)
    tbl_v[pl.ds(N_SP, LANES)] = jnp.zeros((LANES,), jnp.float32)
    pad = jnp.full((LANES,), N_SP, jnp.int32)

    @pl.when(jnp.logical_not(is_tail))
    def _():
        pltpu.async_copy(idx_hbm.at[pl.ds(base, CHUNK)],
                         idx_v.at[pl.ds(0, CHUNK)], sem1).wait()
        idx_v[pl.ds(CHUNK, LANES)] = pad

    @pl.when(is_tail)
    def _():
        pltpu.async_copy(idx_hbm.at[pl.ds(base, TAIL)],
                         idx_v.at[pl.ds(0, TAIL)], sem1).wait()
        for i in range(TAIL // LANES, STEPS):
            idx_v[pl.ds(i * LANES, LANES)] = pad

    tbl_cp.wait()
    zero = jnp.zeros((LANES,), jnp.float32)

    def step4(j, accs):
        a0, a1, a2, a3 = accs
        b = j * (ACCS * LANES)
        a0 = a0 + plsc.load_gather(tbl_v, [idx_v[pl.ds(b, LANES)]])
        a1 = a1 + plsc.load_gather(tbl_v, [idx_v[pl.ds(b + LANES, LANES)]])
        a2 = a2 + plsc.load_gather(tbl_v, [idx_v[pl.ds(b + 2 * LANES, LANES)]])
        a3 = a3 + plsc.load_gather(tbl_v, [idx_v[pl.ds(b + 3 * LANES, LANES)]])
        return a0, a1, a2, a3

    accs = lax.fori_loop(0, STEPS // ACCS, step4, (zero, zero, zero, zero),
                         unroll=2)
    vec_v[...] = (accs[0] + accs[1]) + (accs[2] + accs[3])
    pltpu.sync_copy(vec_v, part_hbm.at[sid])


def _tc_finish(part_ref, te_ref, ts_ref, out_ref):
    s = jnp.sum(part_ref[...]) + ts_ref[0, 0]
    out_ref[...] = te_ref[...] + s


@jax.jit
def _shift_sum(idx, shifts, te_row, ts):
    mesh = plsc.VectorSubcoreMesh(core_axis_name="c", subcore_axis_name="s",
                                  num_cores=1)
    partials = pl.kernel(
        _sc_body,
        out_type=jax.ShapeDtypeStruct((NSUB, LANES), jnp.float32),
        mesh=mesh,
        compiler_params=pltpu.CompilerParams(needs_layout_passes=False),
        scratch_types=[
            pltpu.VMEM((ALLOC,), jnp.int32),
            pltpu.VMEM((TBL,), jnp.float32),
            pltpu.VMEM((LANES,), jnp.float32),
            pltpu.SemaphoreType.DMA,
            pltpu.SemaphoreType.DMA,
        ],
    )(idx, shifts)
    return pl.pallas_call(
        _tc_finish,
        out_shape=jax.ShapeDtypeStruct((1, LANES), jnp.float32),
    )(partials, te_row, ts)


def kernel(total_energy, species_index, shifts, total_shift):
    idx = species_index.astype(jnp.int32)
    te_row = total_energy.reshape(1, LANES)
    ts = total_shift.astype(jnp.float32).reshape(1, 1)
    out = _shift_sum(idx, shifts, te_row, ts)
    return out.reshape(total_energy.shape)
